# baseline (speedup 1.0000x reference)
import functools
import math

import jax
import jax.numpy as jnp
from jax import lax
from jax.experimental import pallas as pl
from jax.experimental.pallas import tpu as pltpu
from jax.experimental.pallas import tpu_sc as plsc

NUM_CORES = 2
NUM_SUBCORES = 16
NUM_WORKERS = NUM_CORES * NUM_SUBCORES
LANES = 16
PADDED_D = 128

SEQ_PAD = 56
NBUF = 4
PREFETCH = 2


@functools.cache
def _make_kernel(BATCH, SEQ, V, D):
    batches_per_w = BATCH // NUM_WORKERS
    n_per_w = batches_per_w * SEQ_PAD
    scale = jnp.float32(math.sqrt(D))

    mesh = plsc.VectorSubcoreMesh(
        core_axis_name="c",
        subcore_axis_name="s",
        num_cores=NUM_CORES,
        num_subcores=NUM_SUBCORES,
    )

    scratch = (
        [pltpu.VMEM((n_per_w,), jnp.int32)]
        + [pltpu.VMEM((SEQ, PADDED_D), jnp.float32) for _ in range(NBUF)]
        + [pltpu.SemaphoreType.DMA for _ in range(2 * NBUF)]
    )

    @functools.partial(
        pl.kernel,
        out_type=jax.ShapeDtypeStruct((BATCH, SEQ, PADDED_D), jnp.float32),
        mesh=mesh,
        scratch_types=scratch,
    )
    def ker(idx_hbm, table_hbm, out_hbm, idx_v, *rest):
        bufs = rest[:NBUF]
        gsems = rest[NBUF : 2 * NBUF]
        ssems = rest[2 * NBUF :]

        wid = lax.axis_index("s") * NUM_CORES + lax.axis_index("c")
        base = wid * n_per_w
        b0 = wid * batches_per_w
        pltpu.sync_copy(idx_hbm.at[pl.ds(base, n_per_w)], idx_v)

        def start_gather(g, b):
            idx_slice = idx_v.at[pl.ds(g * SEQ_PAD, SEQ)]
            pltpu.make_async_copy(table_hbm.at[idx_slice], bufs[b], gsems[b]).start()

        def wait_gather(b):
            pltpu.make_async_copy(
                table_hbm.at[idx_v.at[pl.ds(0, SEQ)]], bufs[b], gsems[b]
            ).wait()

        def start_scatter(g, b):
            pltpu.make_async_copy(bufs[b], out_hbm.at[b0 + g], ssems[b]).start()

        def wait_scatter(b):
            pltpu.make_async_copy(bufs[b], out_hbm.at[b0], ssems[b]).wait()

        for g in range(PREFETCH):
            start_gather(g, g % NBUF)

        @pl.loop(0, batches_per_w, step=NBUF)
        def outer(g0):
            for db in range(NBUF):
                g = g0 + db
                b = db
                bn = (db + PREFETCH) % NBUF

                @pl.when(g + PREFETCH - NBUF >= 0)
                def _():
                    wait_scatter(bn)

                @pl.when(g + PREFETCH < batches_per_w)
                def _():
                    start_gather(g + PREFETCH, bn)

                wait_gather(b)

                def row_body(i, carry):
                    for j in range(D // LANES):
                        bufs[b][i, pl.ds(j * LANES, LANES)] = (
                            bufs[b][i, pl.ds(j * LANES, LANES)] * scale
                        )
                    return carry

                lax.fori_loop(0, SEQ, row_body, 0, unroll=5)
                start_scatter(g, b)

        for g in range(max(0, batches_per_w - (NBUF - PREFETCH)), batches_per_w):
            wait_scatter(g % NBUF)

    return ker


def kernel(sen, table):
    B, L = sen.shape
    V, D = table.shape
    idx = jnp.pad(sen, ((0, 0), (0, SEQ_PAD - L))).reshape(-1)
    t128 = jnp.zeros((V, PADDED_D), jnp.float32).at[:, :D].set(table)
    out = _make_kernel(B, L, V, D)(idx, t128)
    return out[:, :, :D]


# speedup vs baseline: 1.4728x; 1.4728x over previous
import functools
import math

import jax
import jax.numpy as jnp
from jax import lax
from jax.experimental import pallas as pl
from jax.experimental.pallas import tpu as pltpu
from jax.experimental.pallas import tpu_sc as plsc

NUM_CORES = 2
NUM_SUBCORES = 16
NUM_WORKERS = NUM_CORES * NUM_SUBCORES
LANES = 16
PADDED_D = 128

SEQ_PAD = 56
NBUF = 4
PREFETCH = 2


@functools.cache
def _make_kernel(BATCH, SEQ, V, D):
    batches_per_w = BATCH // NUM_WORKERS
    n_per_w = batches_per_w * SEQ_PAD
    scale = jnp.float32(math.sqrt(D))

    mesh = plsc.VectorSubcoreMesh(
        core_axis_name="c",
        subcore_axis_name="s",
        num_cores=NUM_CORES,
        num_subcores=NUM_SUBCORES,
    )

    scratch = (
        [pltpu.VMEM((n_per_w,), jnp.int32)]
        + [pltpu.VMEM((SEQ, PADDED_D), jnp.float32) for _ in range(NBUF)]
        + [pltpu.SemaphoreType.DMA for _ in range(2 * NBUF)]
    )

    @functools.partial(
        pl.kernel,
        out_type=jax.ShapeDtypeStruct((BATCH, SEQ, PADDED_D), jnp.float32),
        mesh=mesh,
        scratch_types=scratch,
    )
    def ker(idx_hbm, table_hbm, out_hbm, idx_v, *rest):
        bufs = rest[:NBUF]
        gsems = rest[NBUF : 2 * NBUF]
        ssems = rest[2 * NBUF :]

        wid = lax.axis_index("s") * NUM_CORES + lax.axis_index("c")
        base = wid * n_per_w
        b0 = wid * batches_per_w
        pltpu.sync_copy(idx_hbm.at[pl.ds(base, n_per_w)], idx_v)

        def start_gather(g, b):
            idx_slice = idx_v.at[pl.ds(g * SEQ_PAD, SEQ)]
            pltpu.make_async_copy(table_hbm.at[idx_slice], bufs[b], gsems[b]).start()

        def wait_gather(b):
            pltpu.make_async_copy(
                table_hbm.at[idx_v.at[pl.ds(0, SEQ)]], bufs[b], gsems[b]
            ).wait()

        def start_scatter(g, b):
            pltpu.make_async_copy(bufs[b], out_hbm.at[b0 + g], ssems[b]).start()

        def wait_scatter(b):
            pltpu.make_async_copy(bufs[b], out_hbm.at[b0], ssems[b]).wait()

        for g in range(PREFETCH):
            start_gather(g, g % NBUF)

        @pl.loop(0, batches_per_w, step=NBUF)
        def outer(g0):
            for db in range(NBUF):
                g = g0 + db
                b = db
                bn = (db + PREFETCH) % NBUF

                @pl.when(g + PREFETCH - NBUF >= 0)
                def _():
                    wait_scatter(bn)

                @pl.when(g + PREFETCH < batches_per_w)
                def _():
                    start_gather(g + PREFETCH, bn)

                wait_gather(b)

                def row_body(i, carry):
                    for j in range(D // LANES):
                        bufs[b][i, pl.ds(j * LANES, LANES)] = (
                            bufs[b][i, pl.ds(j * LANES, LANES)] * scale
                        )
                    return carry

                lax.fori_loop(0, SEQ, row_body, 0, unroll=5)
                start_scatter(g, b)

        for g in range(max(0, batches_per_w - (NBUF - PREFETCH)), batches_per_w):
            wait_scatter(g % NBUF)

    return ker


def kernel(sen, table):
    B, L = sen.shape
    V, D = table.shape
    idx = jnp.pad(sen, ((0, 0), (0, SEQ_PAD - L))).reshape(-1)
    t128 = jnp.pad(table, ((0, 0), (0, PADDED_D - D)))
    out = _make_kernel(B, L, V, D)(idx, t128)
    return out[:, :, :D]
